# baseline (speedup 1.0000x reference)
import functools

import jax
import jax.numpy as jnp
from jax import lax
from jax.experimental import pallas as pl
from jax.experimental.pallas import tpu as pltpu
from jax.experimental.pallas import tpu_sc as plsc

NC = 2
NS = 16
NW = NC * NS
L = 16

IDX_PER_STREAM = 128


@functools.lru_cache(maxsize=None)
def _build(B, F, V, D):
    N = B * F
    assert N % NW == 0
    rpw = N // NW
    assert rpw % IDX_PER_STREAM == 0
    ns = rpw // IDX_PER_STREAM
    assert rpw % F == 0
    spc = 13 if ns % 13 == 0 else 1
    nch = ns // spc
    chunk = spc * IDX_PER_STREAM

    mesh = plsc.VectorSubcoreMesh(core_axis_name="c", subcore_axis_name="s")

    @functools.partial(
        pl.kernel,
        out_type=jax.ShapeDtypeStruct((N, D), jnp.float32),
        mesh=mesh,
        compiler_params=pltpu.CompilerParams(use_tc_tiling_on_sc=False),
        scratch_types=[
            pltpu.VMEM((ns, IDX_PER_STREAM), jnp.int32),
            pltpu.VMEM((chunk, D), jnp.float32),
            pltpu.SemaphoreType.DMA,
        ],
    )
    def gather_kernel(cat_hbm, tab_hbm, out_hbm, idx_v, rows_v, sem):
        cid = lax.axis_index("c")
        sid = lax.axis_index("s")
        wid = sid * NC + cid
        base = wid * rpw

        pltpu.sync_copy(cat_hbm.at[pl.ds(wid * ns, ns)], idx_v)

        lanes = lax.iota(jnp.int32, L)

        def add_off(k, carry):
            for j in range(IDX_PER_STREAM // L):
                pos = k * IDX_PER_STREAM + j * L + lanes
                f = lax.rem(pos, F)
                idx_v[k, pl.ds(j * L, L)] = idx_v[k, pl.ds(j * L, L)] + f * V
            return carry

        lax.fori_loop(0, ns, add_off, 0)

        def do_chunk(i, carry):
            k0 = i * spc
            cps = []
            for j in range(spc):
                cp = pltpu.async_copy(
                    tab_hbm.at[idx_v.at[k0 + j]],
                    rows_v.at[pl.ds(j * IDX_PER_STREAM, IDX_PER_STREAM)],
                    sem,
                )
                cps.append(cp)
            for cp in cps:
                cp.wait()
            pltpu.sync_copy(rows_v, out_hbm.at[pl.ds(base + i * chunk, chunk)])
            return carry

        lax.fori_loop(0, nch, do_chunk, 0)

    return gather_kernel


def kernel(categorical_features, tables):
    B, F = categorical_features.shape
    Ft, V, D = tables.shape
    assert Ft == F
    cat = categorical_features.astype(jnp.int32).reshape(
        B * F // IDX_PER_STREAM, IDX_PER_STREAM)
    tab = tables.reshape(F * V, D)
    out = _build(B, F, V, D)(cat, tab)
    return out.reshape(B, F * D)


# speedup vs baseline: 2.9321x; 2.9321x over previous
import functools

import jax
import jax.numpy as jnp
from jax import lax
from jax.experimental import pallas as pl
from jax.experimental.pallas import tpu as pltpu
from jax.experimental.pallas import tpu_sc as plsc

NC = 2
NS = 16
NW = NC * NS
L = 16

BC = 8192


@functools.lru_cache(maxsize=None)
def _build(B, F, V, D):
    assert D == NW, "one embedding dim per vector subcore"
    assert B % BC == 0 and BC % L == 0
    nbc = B // BC

    mesh = plsc.VectorSubcoreMesh(core_axis_name="c", subcore_axis_name="s")

    @functools.partial(
        pl.kernel,
        out_type=jax.ShapeDtypeStruct((F * D, B), jnp.float32),
        mesh=mesh,
        compiler_params=pltpu.CompilerParams(needs_layout_passes=False),
        scratch_types=[
            pltpu.VMEM((V,), jnp.float32),
            pltpu.VMEM((BC,), jnp.int32),
            pltpu.VMEM((BC,), jnp.float32),
        ],
    )
    def col_gather(cat_hbm, tab_hbm, out_hbm, slab_v, idx_v, res_v):
        cid = lax.axis_index("c")
        sid = lax.axis_index("s")
        d = sid * NC + cid

        def do_field(f, carry):
            pltpu.sync_copy(tab_hbm.at[f, d, :], slab_v)

            def do_chunk(c, carry2):
                pltpu.sync_copy(cat_hbm.at[f, pl.ds(c * BC, BC)], idx_v)

                def gather16(k, carry3):
                    iv = idx_v[pl.ds(k * L, L)]
                    res_v[pl.ds(k * L, L)] = plsc.load_gather(slab_v, [iv])
                    return carry3

                lax.fori_loop(0, BC // L, gather16, 0, unroll=8)
                pltpu.sync_copy(
                    res_v, out_hbm.at[f * D + d, pl.ds(c * BC, BC)])
                return carry2

            lax.fori_loop(0, nbc, do_chunk, 0)
            return carry

        lax.fori_loop(0, F, do_field, 0)

    return col_gather


def kernel(categorical_features, tables):
    B, F = categorical_features.shape
    Ft, V, D = tables.shape
    assert Ft == F
    cat_t = categorical_features.astype(jnp.int32).T
    tab_t = jnp.transpose(tables, (0, 2, 1))
    out_t = _build(B, F, V, D)(cat_t, tab_t)
    return out_t.T


# speedup vs baseline: 4.3580x; 1.4863x over previous
import functools

import jax
import jax.numpy as jnp
from jax import lax
from jax.experimental import pallas as pl
from jax.experimental.pallas import tpu as pltpu
from jax.experimental.pallas import tpu_sc as plsc

NC = 2
NS = 16
NW = NC * NS
L = 16


@functools.lru_cache(maxsize=None)
def _build(B, F, V, D):
    assert D == NW, "one embedding dim per vector subcore"
    assert B % L == 0

    mesh = plsc.VectorSubcoreMesh(core_axis_name="c", subcore_axis_name="s")

    @functools.partial(
        pl.kernel,
        out_type=jax.ShapeDtypeStruct((F * D, B), jnp.float32),
        mesh=mesh,
        compiler_params=pltpu.CompilerParams(needs_layout_passes=False),
        scratch_types=[
            pltpu.VMEM((V,), jnp.float32),
            pltpu.VMEM((B,), jnp.float32),
            pltpu.SemaphoreType.DMA,
        ],
    )
    def col_gather(cat_hbm, tab_hbm, out_hbm, slab_v, buf_v, sem):
        cid = lax.axis_index("c")
        sid = lax.axis_index("s")
        d = sid * NC + cid

        pltpu.async_copy(tab_hbm.at[0, d, :], slab_v, sem)

        def do_field(f, carry):
            pltpu.sync_copy(cat_hbm.at[f, :], buf_v)
            pltpu.make_async_copy(tab_hbm.at[f, d, :], slab_v, sem).wait()

            def gather16(k, carry2):
                iv = plsc.bitcast(buf_v[pl.ds(k * L, L)], jnp.int32)
                buf_v[pl.ds(k * L, L)] = plsc.load_gather(slab_v, [iv])
                return carry2

            lax.fori_loop(0, B // L, gather16, 0, unroll=16)

            @pl.when(f < F - 1)
            def _():
                pltpu.async_copy(tab_hbm.at[f + 1, d, :], slab_v, sem)

            pltpu.sync_copy(buf_v, out_hbm.at[f * D + d, :])
            return carry

        lax.fori_loop(0, F, do_field, 0)

    return col_gather


def kernel(categorical_features, tables):
    B, F = categorical_features.shape
    Ft, V, D = tables.shape
    assert Ft == F
    cat_f32 = lax.bitcast_convert_type(
        categorical_features.astype(jnp.int32), jnp.float32)
    cat_t = cat_f32.T
    tab_t = jnp.transpose(tables, (0, 2, 1))
    out_t = _build(B, F, V, D)(cat_t, tab_t)
    return out_t.T


# speedup vs baseline: 4.3908x; 1.0075x over previous
import functools

import jax
import jax.numpy as jnp
from jax import lax
from jax.experimental import pallas as pl
from jax.experimental.pallas import tpu as pltpu
from jax.experimental.pallas import tpu_sc as plsc

NC = 2
NS = 16
NW = NC * NS
L = 16


@functools.lru_cache(maxsize=None)
def _build(B, F, V, D):
    assert D == NW, "one embedding dim per vector subcore"
    assert B % L == 0

    mesh = plsc.VectorSubcoreMesh(core_axis_name="c", subcore_axis_name="s")

    @functools.partial(
        pl.kernel,
        out_type=jax.ShapeDtypeStruct((F * D, B), jnp.float32),
        mesh=mesh,
        compiler_params=pltpu.CompilerParams(needs_layout_passes=False),
        scratch_types=[
            pltpu.VMEM((V,), jnp.float32),
            pltpu.VMEM((B,), jnp.float32),
            pltpu.SemaphoreType.DMA,
            pltpu.SemaphoreType.DMA,
        ],
    )
    def col_gather(cat_hbm, tab_hbm, out_hbm, slab_v, buf_v, sem, semo):
        cid = lax.axis_index("c")
        sid = lax.axis_index("s")
        d = sid * NC + cid
        BH = B // 2

        pltpu.async_copy(tab_hbm.at[0, d, :], slab_v, sem)

        def gather16(k, carry2):
            iv = plsc.bitcast(buf_v[pl.ds(k * L, L)], jnp.int32)
            buf_v[pl.ds(k * L, L)] = plsc.load_gather(slab_v, [iv])
            return carry2

        def do_field(f, carry):
            @pl.when(f > 0)
            def _():
                pltpu.make_async_copy(
                    buf_v, out_hbm.at[(f - 1) * D + d, :], semo).wait()

            pltpu.sync_copy(cat_hbm.at[f, :], buf_v)
            pltpu.make_async_copy(tab_hbm.at[f, d, :], slab_v, sem).wait()

            lax.fori_loop(0, BH // L, gather16, 0, unroll=16)
            pltpu.async_copy(
                buf_v.at[pl.ds(0, BH)],
                out_hbm.at[f * D + d, pl.ds(0, BH)], semo)
            lax.fori_loop(BH // L, B // L, gather16, 0, unroll=16)

            @pl.when(f < F - 1)
            def _():
                pltpu.async_copy(tab_hbm.at[f + 1, d, :], slab_v, sem)

            pltpu.async_copy(
                buf_v.at[pl.ds(BH, B - BH)],
                out_hbm.at[f * D + d, pl.ds(BH, B - BH)], semo)
            return carry

        lax.fori_loop(0, F, do_field, 0)
        pltpu.make_async_copy(
            buf_v, out_hbm.at[(F - 1) * D + d, :], semo).wait()

    return col_gather


def kernel(categorical_features, tables):
    B, F = categorical_features.shape
    Ft, V, D = tables.shape
    assert Ft == F
    cat_f32 = lax.bitcast_convert_type(
        categorical_features.astype(jnp.int32), jnp.float32)
    cat_t = cat_f32.T
    tab_t = jnp.transpose(tables, (0, 2, 1))
    out_t = _build(B, F, V, D)(cat_t, tab_t)
    return out_t.T


# speedup vs baseline: 4.4212x; 1.0069x over previous
import functools

import jax
import jax.numpy as jnp
from jax import lax
from jax.experimental import pallas as pl
from jax.experimental.pallas import tpu as pltpu
from jax.experimental.pallas import tpu_sc as plsc

NC = 2
NS = 16
NW = NC * NS
L = 16


@functools.lru_cache(maxsize=None)
def _build(B, F, V, D):
    assert D == NW, "one embedding dim per vector subcore"
    BH = B // 2
    assert BH % L == 0

    mesh = plsc.VectorSubcoreMesh(core_axis_name="c", subcore_axis_name="s")

    @functools.partial(
        pl.kernel,
        out_type=jax.ShapeDtypeStruct((F * D, B), jnp.float32),
        mesh=mesh,
        compiler_params=pltpu.CompilerParams(needs_layout_passes=False),
        scratch_types=[
            pltpu.VMEM((V,), jnp.float32),
            pltpu.VMEM((BH,), jnp.float32),
            pltpu.VMEM((BH,), jnp.float32),
            pltpu.SemaphoreType.DMA,
            pltpu.SemaphoreType.DMA,
            pltpu.SemaphoreType.DMA,
            pltpu.SemaphoreType.DMA,
            pltpu.SemaphoreType.DMA,
        ],
    )
    def col_gather(cat_hbm, tab_hbm, out_hbm, slab_v, buf_a, buf_b,
                   sem_s, sem_ia, sem_ib, sem_oa, sem_ob):
        cid = lax.axis_index("c")
        sid = lax.axis_index("s")
        d = sid * NC + cid

        def idx_src(f, h):
            return cat_hbm.at[f, pl.ds(h * BH, BH)]

        def out_dst(f, h):
            return out_hbm.at[f * D + d, pl.ds(h * BH, BH)]

        def gather_half(buf):
            def gather16(k, carry):
                iv = plsc.bitcast(buf[pl.ds(k * L, L)], jnp.int32)
                buf[pl.ds(k * L, L)] = plsc.load_gather(slab_v, [iv])
                return carry

            lax.fori_loop(0, BH // L, gather16, 0, unroll=16)

        pltpu.async_copy(tab_hbm.at[0, d, :], slab_v, sem_s)
        pltpu.async_copy(idx_src(0, 0), buf_a, sem_ia)

        def do_field(f, carry):
            @pl.when(f > 0)
            def _():
                pltpu.make_async_copy(buf_b, out_dst(f - 1, 1), sem_ob).wait()
            pltpu.async_copy(idx_src(f, 1), buf_b, sem_ib)

            pltpu.make_async_copy(idx_src(f, 0), buf_a, sem_ia).wait()
            pltpu.make_async_copy(tab_hbm.at[f, d, :], slab_v, sem_s).wait()
            gather_half(buf_a)
            pltpu.async_copy(buf_a, out_dst(f, 0), sem_oa)

            pltpu.make_async_copy(idx_src(f, 1), buf_b, sem_ib).wait()
            gather_half(buf_b)

            @pl.when(f < F - 1)
            def _():
                pltpu.async_copy(tab_hbm.at[f + 1, d, :], slab_v, sem_s)
            pltpu.async_copy(buf_b, out_dst(f, 1), sem_ob)

            pltpu.make_async_copy(buf_a, out_dst(f, 0), sem_oa).wait()

            @pl.when(f < F - 1)
            def _():
                pltpu.async_copy(idx_src(f + 1, 0), buf_a, sem_ia)
            return carry

        lax.fori_loop(0, F, do_field, 0)
        pltpu.make_async_copy(buf_b, out_dst(F - 1, 1), sem_ob).wait()

    return col_gather


def kernel(categorical_features, tables):
    B, F = categorical_features.shape
    Ft, V, D = tables.shape
    assert Ft == F
    cat_f32 = lax.bitcast_convert_type(
        categorical_features.astype(jnp.int32), jnp.float32)
    cat_t = cat_f32.T
    tab_t = jnp.transpose(tables, (0, 2, 1))
    out_t = _build(B, F, V, D)(cat_t, tab_t)
    return out_t.T
